# Initial kernel scaffold; baseline (speedup 1.0000x reference)
import jax
import jax.numpy as jnp
from jax.experimental import pallas as pl


def kernel(input):
    raise NotImplementedError("write your pallas kernel here")


# speedup vs baseline: 1.3146x; 1.3146x over previous
import jax
import jax.numpy as jnp
from jax.experimental import pallas as pl
from jax.experimental.pallas import tpu as pltpu

_B, _H, _W = 16, 512, 512
_N = _B * _H * _W


def _sum_body(x_ref, s_ref):
    i = pl.program_id(0)

    @pl.when(i == 0)
    def _init():
        s_ref[0, 0] = jnp.float32(0.0)

    s_ref[0, 0] += jnp.sum(x_ref[...])


def _mask_body(s_ref, x_ref, m_ref, c_ref):
    i = pl.program_id(0)
    x = x_ref[0]
    ninf = jnp.float32(-jnp.inf)
    col = jax.lax.broadcasted_iota(jnp.int32, (_H, _W), 1)
    row = jax.lax.broadcasted_iota(jnp.int32, (_H, _W), 0)
    m = jnp.maximum(
        jnp.maximum(
            jnp.where(col > 0, pltpu.roll(x, 1, 1), ninf),
            jnp.where(col < _W - 1, pltpu.roll(x, _W - 1, 1), ninf),
        ),
        x,
    )
    pooled = jnp.maximum(
        jnp.maximum(
            jnp.where(row > 0, pltpu.roll(m, 1, 0), ninf),
            jnp.where(row < _H - 1, pltpu.roll(m, _H - 1, 0), ninf),
        ),
        m,
    )
    mean = s_ref[0, 0] * jnp.float32(1.0 / _N)
    mask = (x == pooled) & (x > mean)
    m_ref[0] = mask

    @pl.when(i == 0)
    def _init():
        c_ref[0, 0] = jnp.int32(0)

    c_ref[0, 0] += jnp.sum(mask.astype(jnp.int32))


def kernel(input):
    x3 = input.reshape(_B, _H, _W)
    s = pl.pallas_call(
        _sum_body,
        grid=(_B,),
        in_specs=[pl.BlockSpec((1, _H, _W), lambda i: (i, 0, 0))],
        out_specs=pl.BlockSpec(memory_space=pltpu.SMEM),
        out_shape=jax.ShapeDtypeStruct((1, 1), jnp.float32),
    )(x3)
    mask, cnt = pl.pallas_call(
        _mask_body,
        grid=(_B,),
        in_specs=[
            pl.BlockSpec(memory_space=pltpu.SMEM),
            pl.BlockSpec((1, _H, _W), lambda i: (i, 0, 0)),
        ],
        out_specs=[
            pl.BlockSpec((1, _H, _W), lambda i: (i, 0, 0)),
            pl.BlockSpec(memory_space=pltpu.SMEM),
        ],
        out_shape=[
            jax.ShapeDtypeStruct((_B, _H, _W), jnp.bool_),
            jax.ShapeDtypeStruct((1, 1), jnp.int32),
        ],
    )(s, x3)
    return mask.reshape(_B, 1, _H, _W), cnt[0, 0]


# speedup vs baseline: 1.6193x; 1.2318x over previous
import jax
import jax.numpy as jnp
from jax.experimental import pallas as pl
from jax.experimental.pallas import tpu as pltpu

_B, _H, _W = 16, 512, 512
_N = _B * _H * _W


def _fused_body(x_ref, m_ref, c_ref, buf_ref, s_ref):
    s = pl.program_id(0)

    @pl.when(s == 0)
    def _init_sum():
        s_ref[0, 0] = jnp.float32(0.0)

    @pl.when(s < _B)
    def _phase1():
        x = x_ref[0]
        ninf = jnp.float32(-jnp.inf)
        col = jax.lax.broadcasted_iota(jnp.int32, (_H, _W), 1)
        row = jax.lax.broadcasted_iota(jnp.int32, (_H, _W), 0)
        m = jnp.maximum(
            jnp.maximum(
                jnp.where(col > 0, pltpu.roll(x, 1, 1), ninf),
                jnp.where(col < _W - 1, pltpu.roll(x, _W - 1, 1), ninf),
            ),
            x,
        )
        pooled = jnp.maximum(
            jnp.maximum(
                jnp.where(row > 0, pltpu.roll(m, 1, 0), ninf),
                jnp.where(row < _H - 1, pltpu.roll(m, _H - 1, 0), ninf),
            ),
            m,
        )
        y = jnp.where(x == pooled, x, ninf)
        buf_ref[pl.ds(s, 1)] = y[None]
        s_ref[0, 0] += jnp.sum(x)

    @pl.when(s >= _B)
    def _phase2():
        i = s - _B
        mean = s_ref[0, 0] * jnp.float32(1.0 / _N)
        y = buf_ref[pl.ds(i, 1)][0]
        mask = y > mean
        m_ref[0] = mask

        @pl.when(s == _B)
        def _init_cnt():
            c_ref[0, 0] = jnp.int32(0)

        c_ref[0, 0] += jnp.sum(mask.astype(jnp.int32))


def kernel(input):
    x3 = input.reshape(_B, _H, _W)
    mask, cnt = pl.pallas_call(
        _fused_body,
        grid=(2 * _B,),
        in_specs=[
            pl.BlockSpec((1, _H, _W), lambda s: (jnp.minimum(s, _B - 1), 0, 0)),
        ],
        out_specs=[
            pl.BlockSpec((1, _H, _W), lambda s: (jnp.maximum(s - _B, 0), 0, 0)),
            pl.BlockSpec(memory_space=pltpu.SMEM),
        ],
        out_shape=[
            jax.ShapeDtypeStruct((_B, _H, _W), jnp.bool_),
            jax.ShapeDtypeStruct((1, 1), jnp.int32),
        ],
        scratch_shapes=[
            pltpu.VMEM((_B, _H, _W), jnp.float32),
            pltpu.SMEM((1, 1), jnp.float32),
        ],
    )(x3)
    return mask.reshape(_B, 1, _H, _W), cnt[0, 0]


# speedup vs baseline: 1.9063x; 1.1773x over previous
import jax
import jax.numpy as jnp
from jax.experimental import pallas as pl
from jax.experimental.pallas import tpu as pltpu

_B, _H, _W = 16, 512, 512
_N = _B * _H * _W
_BB = 4
_S = _B // _BB


def _fused_body(x_ref, m_ref, c_ref, buf_ref, s_ref):
    s = pl.program_id(0)

    @pl.when(s == 0)
    def _init_sum():
        s_ref[0, 0] = jnp.float32(0.0)

    @pl.when(s < _S)
    def _phase1():
        x = x_ref[...]
        ninf = jnp.float32(-jnp.inf)
        col = jax.lax.broadcasted_iota(jnp.int32, (_BB, _H, _W), 2)
        row = jax.lax.broadcasted_iota(jnp.int32, (_BB, _H, _W), 1)
        m = jnp.maximum(
            jnp.maximum(
                jnp.where(col > 0, pltpu.roll(x, 1, 2), ninf),
                jnp.where(col < _W - 1, pltpu.roll(x, _W - 1, 2), ninf),
            ),
            x,
        )
        pooled = jnp.maximum(
            jnp.maximum(
                jnp.where(row > 0, pltpu.roll(m, 1, 1), ninf),
                jnp.where(row < _H - 1, pltpu.roll(m, _H - 1, 1), ninf),
            ),
            m,
        )
        y = jnp.where(x == pooled, x, ninf)
        buf_ref[pl.ds(s * _BB, _BB)] = y
        s_ref[0, 0] += jnp.sum(x)

    @pl.when(s >= _S)
    def _phase2():
        i = s - _S
        mean = s_ref[0, 0] * jnp.float32(1.0 / _N)
        y = buf_ref[pl.ds(i * _BB, _BB)]
        mask = y > mean
        m_ref[...] = mask

        @pl.when(s == _S)
        def _init_cnt():
            c_ref[0, 0] = jnp.int32(0)

        c_ref[0, 0] += jnp.sum(mask.astype(jnp.int32))


def kernel(input):
    x3 = input.reshape(_B, _H, _W)
    mask, cnt = pl.pallas_call(
        _fused_body,
        grid=(2 * _S,),
        in_specs=[
            pl.BlockSpec((_BB, _H, _W), lambda s: (jnp.minimum(s, _S - 1), 0, 0)),
        ],
        out_specs=[
            pl.BlockSpec((_BB, _H, _W), lambda s: (jnp.maximum(s - _S, 0), 0, 0)),
            pl.BlockSpec(memory_space=pltpu.SMEM),
        ],
        out_shape=[
            jax.ShapeDtypeStruct((_B, _H, _W), jnp.bool_),
            jax.ShapeDtypeStruct((1, 1), jnp.int32),
        ],
        scratch_shapes=[
            pltpu.VMEM((_B, _H, _W), jnp.float32),
            pltpu.SMEM((1, 1), jnp.float32),
        ],
    )(x3)
    return mask.reshape(_B, 1, _H, _W), cnt[0, 0]


# speedup vs baseline: 1.9117x; 1.0028x over previous
import jax
import jax.numpy as jnp
from jax.experimental import pallas as pl
from jax.experimental.pallas import tpu as pltpu

_B, _H, _W = 16, 512, 512
_N = _B * _H * _W
_BB = 4
_S = _B // _BB


def _fused_body(x_hbm, m_ref, c_ref, land, buf, s_ref, in_sems):
    s = pl.program_id(0)

    @pl.when(s == 0)
    def _prologue():
        s_ref[0, 0] = jnp.float32(0.0)
        pltpu.make_async_copy(
            x_hbm.at[pl.ds(0, _BB)], land.at[0], in_sems.at[0]
        ).start()
        pltpu.make_async_copy(
            x_hbm.at[pl.ds(_BB, _BB)], land.at[1], in_sems.at[1]
        ).start()

    @pl.when(s < _S)
    def _phase1():
        slot = jax.lax.rem(s, 2)
        pltpu.make_async_copy(
            x_hbm.at[pl.ds(s * _BB, _BB)], land.at[slot], in_sems.at[slot]
        ).wait()

        x = land[slot]
        ninf = jnp.float32(-jnp.inf)
        col = jax.lax.broadcasted_iota(jnp.int32, (_BB, _H, _W), 2)
        row = jax.lax.broadcasted_iota(jnp.int32, (_BB, _H, _W), 1)
        m = jnp.maximum(
            jnp.maximum(
                jnp.where(col > 0, pltpu.roll(x, 1, 2), ninf),
                jnp.where(col < _W - 1, pltpu.roll(x, _W - 1, 2), ninf),
            ),
            x,
        )
        pooled = jnp.maximum(
            jnp.maximum(
                jnp.where(row > 0, pltpu.roll(m, 1, 1), ninf),
                jnp.where(row < _H - 1, pltpu.roll(m, _H - 1, 1), ninf),
            ),
            m,
        )
        buf[pl.ds(s * _BB, _BB)] = jnp.where(x == pooled, x, ninf)
        s_ref[0, 0] += jnp.sum(x)

        @pl.when(s + 2 < _S)
        def _prefetch():
            pltpu.make_async_copy(
                x_hbm.at[pl.ds((s + 2) * _BB, _BB)],
                land.at[slot],
                in_sems.at[slot],
            ).start()

    @pl.when(s >= _S)
    def _phase2():
        i = s - _S
        mean = s_ref[0, 0] * jnp.float32(1.0 / _N)
        y = buf[pl.ds(i * _BB, _BB)]
        mask = y > mean
        m_ref[...] = mask

        @pl.when(s == _S)
        def _init_cnt():
            c_ref[0, 0] = jnp.int32(0)

        c_ref[0, 0] += jnp.sum(mask.astype(jnp.int32))


def kernel(input):
    x3 = input.reshape(_B, _H, _W)
    mask, cnt = pl.pallas_call(
        _fused_body,
        grid=(2 * _S,),
        in_specs=[pl.BlockSpec(memory_space=pl.ANY)],
        out_specs=[
            pl.BlockSpec((_BB, _H, _W), lambda s: (jnp.maximum(s - _S, 0), 0, 0)),
            pl.BlockSpec(memory_space=pltpu.SMEM),
        ],
        out_shape=[
            jax.ShapeDtypeStruct((_B, _H, _W), jnp.bool_),
            jax.ShapeDtypeStruct((1, 1), jnp.int32),
        ],
        scratch_shapes=[
            pltpu.VMEM((2, _BB, _H, _W), jnp.float32),
            pltpu.VMEM((_B, _H, _W), jnp.float32),
            pltpu.SMEM((1, 1), jnp.float32),
            pltpu.SemaphoreType.DMA((2,)),
        ],
    )(x3)
    return mask.reshape(_B, 1, _H, _W), cnt[0, 0]
